# baseline (speedup 1.0000x reference)
import jax
import jax.numpy as jnp
from jax.experimental import pallas as pl

_F32 = jnp.float32
_BF16 = jnp.bfloat16
_G = 4


def _dot_tn(a, b):
    return jax.lax.dot_general(
        a, b, (((0,), (0,)), ((), ())), preferred_element_type=_F32)


def _dot_nt(a, b):
    return jax.lax.dot_general(
        a, b, (((1,), (1,)), ((), ())), preferred_element_type=_F32)


def _dot(a, b):
    return jnp.dot(a, b, preferred_element_type=_F32)


def _col(row):
    return jnp.transpose(row)


def _mlp2_t(x, W1, b1, W2, b2):
    return _dot_tn(W2, jnp.maximum(_dot_tn(W1, x) + _col(b1), 0.0)) + _col(b2)


def _lpgcn_body(A_ref, c_ref, b_ref, cons_ref, l_ref, u_ref, *refs):
    out_ref = refs[-1]
    wrefs = refs[:-1]

    def w(i):
        return wrefs[i][...]

    m = A_ref.shape[1]
    n = A_ref.shape[2]

    Ehs = []
    Els = []
    for g in range(_G):
        Ef = A_ref[g]
        Eh = Ef.astype(_BF16)
        Ehs.append(Eh)
        Els.append((Ef - Eh.astype(_F32)).astype(_BF16))
    hv = jnp.concatenate(
        [jnp.concatenate([b_ref[g], cons_ref[g]], axis=0) for g in range(_G)],
        axis=1)
    hw = jnp.concatenate(
        [jnp.concatenate([c_ref[g], l_ref[g], u_ref[g]], axis=0)
         for g in range(_G)], axis=1)
    hv = _mlp2_t(hv, w(0), w(1), w(2), w(3))
    hw = _mlp2_t(hw, w(4), w(5), w(6), w(7))

    k = 8
    for lyr in range(4):
        Wr, Wm, bh, Wo, bo = (w(k + j) for j in range(5))
        Wr2, Wm2, bh2, Wo2, bo2 = (w(k + 20 + j) for j in range(5))
        k += 5
        q = _dot_tn(Wm, hw).astype(_BF16)
        p = _dot_tn(Wm2, hv).astype(_BF16)
        mv = jnp.concatenate(
            [_dot_nt(q[:, g * n:(g + 1) * n], Ehs[g])
             + _dot_nt(q[:, g * n:(g + 1) * n], Els[g]) for g in range(_G)],
            axis=1)
        mw = jnp.concatenate(
            [_dot(p[:, g * m:(g + 1) * m], Ehs[g])
             + _dot(p[:, g * m:(g + 1) * m], Els[g]) for g in range(_G)],
            axis=1)
        hv = _dot_tn(Wo, jnp.maximum(
            _dot_tn(Wr, hv) + mv + _col(bh), 0.0)) + _col(bo)
        hw = _dot_tn(Wo2, jnp.maximum(
            _dot_tn(Wr2, hw) + mw + _col(bh2), 0.0)) + _col(bo2)

    for g in range(_G):
        pooled = jnp.concatenate(
            [jnp.sum(hv[:, g * m:(g + 1) * m], axis=1, keepdims=True),
             jnp.sum(hw[:, g * n:(g + 1) * n], axis=1, keepdims=True)],
            axis=0)
        res = _mlp2_t(pooled, w(48), w(49), w(50), w(51))
        out_ref[g] = jnp.broadcast_to(res, (1, 128))


def kernel(c, A, b, constraints, l, u, edge_index, phi, params):
    B, m, n = A.shape

    def prep(seq):
        return [a.reshape(1, -1) if a.ndim == 1 else a for a in seq]

    wl = prep(params['fv_in']) + prep(params['fw_in'])
    for lyr in range(4):
        wl += prep(params['cv'][lyr])
    for lyr in range(4):
        wl += prep(params['cw'][lyr])
    wl += prep(params['f_out'])

    vecs = [c.reshape(B, 1, n), b.reshape(B, 1, m),
            constraints.reshape(B, 1, m), l.reshape(B, 1, n),
            u.reshape(B, 1, n)]

    batchspec = lambda shape: pl.BlockSpec((_G,) + shape[1:],
                                           lambda i: (i, 0, 0))
    wspec = lambda a: pl.BlockSpec(a.shape, lambda i: (0, 0))

    out = pl.pallas_call(
        _lpgcn_body,
        grid=(B // _G,),
        in_specs=[batchspec(A.shape)] + [batchspec(v.shape) for v in vecs]
                 + [wspec(a) for a in wl],
        out_specs=pl.BlockSpec((_G, 1, 128), lambda i: (i, 0, 0)),
        out_shape=jax.ShapeDtypeStruct((B, 1, 128), _F32),
    )(A, *vecs, *wl)
    return out[:, 0, :1]


# speedup vs baseline: 1.2564x; 1.2564x over previous
import jax
import jax.numpy as jnp
from jax.experimental import pallas as pl

_F32 = jnp.float32
_BF16 = jnp.bfloat16
_G = 4


def _dot_tn(a, b):
    return jax.lax.dot_general(
        a, b, (((0,), (0,)), ((), ())), preferred_element_type=_F32)


def _dot_nt(a, b):
    return jax.lax.dot_general(
        a, b, (((1,), (1,)), ((), ())), preferred_element_type=_F32)


def _dot(a, b):
    return jnp.dot(a, b, preferred_element_type=_F32)


def _col(row):
    return jnp.transpose(row)


def _mlp2_t(x, W1, b1, W2, b2):
    return _dot_tn(W2, jnp.maximum(_dot_tn(W1, x) + _col(b1), 0.0)) + _col(b2)


def _lpgcn_body(A_ref, c_ref, b_ref, cons_ref, l_ref, u_ref, *refs):
    out_ref = refs[-1]
    wrefs = refs[:-1]

    def w(i):
        return wrefs[i][...]

    m = A_ref.shape[1]
    n = A_ref.shape[2]

    Es = [A_ref[g].astype(_BF16) for g in range(_G)]
    hv = jnp.concatenate(
        [jnp.concatenate([b_ref[g], cons_ref[g]], axis=0) for g in range(_G)],
        axis=1)
    hw = jnp.concatenate(
        [jnp.concatenate([c_ref[g], l_ref[g], u_ref[g]], axis=0)
         for g in range(_G)], axis=1)
    hv = _mlp2_t(hv, w(0), w(1), w(2), w(3))
    hw = _mlp2_t(hw, w(4), w(5), w(6), w(7))

    k = 8
    for lyr in range(4):
        Wr, Wm, bh, Wo, bo = (w(k + j) for j in range(5))
        Wr2, Wm2, bh2, Wo2, bo2 = (w(k + 20 + j) for j in range(5))
        k += 5
        q = _dot_tn(Wm, hw).astype(_BF16)
        p = _dot_tn(Wm2, hv).astype(_BF16)
        mv = jnp.concatenate(
            [_dot_nt(q[:, g * n:(g + 1) * n], Es[g]) for g in range(_G)],
            axis=1)
        mw = jnp.concatenate(
            [_dot(p[:, g * m:(g + 1) * m], Es[g]) for g in range(_G)],
            axis=1)
        hv = _dot_tn(Wo, jnp.maximum(
            _dot_tn(Wr, hv) + mv + _col(bh), 0.0)) + _col(bo)
        hw = _dot_tn(Wo2, jnp.maximum(
            _dot_tn(Wr2, hw) + mw + _col(bh2), 0.0)) + _col(bo2)

    for g in range(_G):
        pooled = jnp.concatenate(
            [jnp.sum(hv[:, g * m:(g + 1) * m], axis=1, keepdims=True),
             jnp.sum(hw[:, g * n:(g + 1) * n], axis=1, keepdims=True)],
            axis=0)
        res = _mlp2_t(pooled, w(48), w(49), w(50), w(51))
        out_ref[g] = jnp.broadcast_to(res, (1, 128))


def kernel(c, A, b, constraints, l, u, edge_index, phi, params):
    B, m, n = A.shape

    def prep(seq):
        return [a.reshape(1, -1) if a.ndim == 1 else a for a in seq]

    wl = prep(params['fv_in']) + prep(params['fw_in'])
    for lyr in range(4):
        wl += prep(params['cv'][lyr])
    for lyr in range(4):
        wl += prep(params['cw'][lyr])
    wl += prep(params['f_out'])

    vecs = [c.reshape(B, 1, n), b.reshape(B, 1, m),
            constraints.reshape(B, 1, m), l.reshape(B, 1, n),
            u.reshape(B, 1, n)]

    batchspec = lambda shape: pl.BlockSpec((_G,) + shape[1:],
                                           lambda i: (i, 0, 0))
    wspec = lambda a: pl.BlockSpec(a.shape, lambda i: (0, 0))

    out = pl.pallas_call(
        _lpgcn_body,
        grid=(B // _G,),
        in_specs=[batchspec(A.shape)] + [batchspec(v.shape) for v in vecs]
                 + [wspec(a) for a in wl],
        out_specs=pl.BlockSpec((_G, 1, 128), lambda i: (i, 0, 0)),
        out_shape=jax.ShapeDtypeStruct((B, 1, 128), _F32),
    )(A, *vecs, *wl)
    return out[:, 0, :1]
